# baseline (speedup 1.0000x reference)
import jax
import jax.numpy as jnp
from jax import lax
from jax.experimental import pallas as pl
from jax.experimental.pallas import tpu as pltpu

B, N, D_IN, D_OUT = 16, 4096, 64, 64


def _fused_body(num_unit_ref, g_ref, feat_ref, v_ref, b_ref, out_ref):
    i = pl.program_id(0)
    nu = num_unit_ref[i]
    x = feat_ref[0]
    row_ids = lax.broadcasted_iota(jnp.int32, (N, D_IN), 0)
    masked = jnp.where(row_ids < nu, x, -jnp.inf)
    fmax = jnp.max(masked, axis=0, keepdims=True)
    h = x - fmax
    v = v_ref[...]
    norm = jnp.sqrt(jnp.sum(v * v))
    w = v * (g_ref[0] / norm)
    out = lax.dot_general(h, w, (((1,), (1,)), ((), ())),
                          preferred_element_type=jnp.float32)
    out_ref[0] = jnp.maximum(out + b_ref[...], 0.0)


def kernel(feat, num_unit, v, g, b):
    g2 = jnp.reshape(g, (1,))
    b2 = jnp.reshape(b, (1, D_OUT))
    grid_spec = pltpu.PrefetchScalarGridSpec(
        num_scalar_prefetch=2,
        grid=(B,),
        in_specs=[
            pl.BlockSpec((1, N, D_IN), lambda i, *_: (i, 0, 0)),
            pl.BlockSpec((D_OUT, D_IN), lambda i, *_: (0, 0)),
            pl.BlockSpec((1, D_OUT), lambda i, *_: (0, 0)),
        ],
        out_specs=pl.BlockSpec((1, N, D_OUT), lambda i, *_: (i, 0, 0)),
    )
    return pl.pallas_call(
        _fused_body,
        grid_spec=grid_spec,
        out_shape=jax.ShapeDtypeStruct((B, N, D_OUT), jnp.float32),
    )(num_unit, g2, feat, v, b2)
